# baseline (speedup 1.0000x reference)
import functools

import jax
import jax.numpy as jnp
from jax.experimental import pallas as pl
from jax.experimental.pallas import tpu as pltpu

_E = 8
_D = 1024
_B = 256


def _plan(actions, n_tokens, n_blocks, n_padded):
    a = actions.astype(jnp.int32)
    order = jnp.argsort(a).astype(jnp.int32)
    sa = a[order]
    counts = jnp.zeros((_E,), jnp.int32).at[a].add(1)
    offsets = jnp.concatenate(
        [jnp.zeros((1,), jnp.int32), jnp.cumsum(counts)[:-1]])
    nblk = (counts + _B - 1) // _B
    cum_nblk = jnp.cumsum(nblk)
    blk_start = jnp.concatenate(
        [jnp.zeros((1,), jnp.int32), cum_nblk[:-1]])
    g = jnp.arange(n_blocks, dtype=jnp.int32)
    block_expert = jnp.minimum(
        jnp.sum((g[:, None] >= cum_nblk[None, :]).astype(jnp.int32), axis=1),
        _E - 1).astype(jnp.int32)
    i = jnp.arange(n_tokens, dtype=jnp.int32)
    dest = blk_start[sa] * _B + (i - offsets[sa])
    src_idx = jnp.zeros((n_padded,), jnp.int32).at[dest].set(order)
    out_idx = jnp.zeros((n_tokens,), jnp.int32).at[order].set(dest)
    return src_idx, out_idx, block_expert


def _gmm_body(be_ref, x_ref, w_ref, b_ref, o_ref):
    o_ref[...] = jax.lax.dot_general(
        x_ref[...], w_ref[0],
        dimension_numbers=(((1,), (1,)), ((), ())),
        preferred_element_type=jnp.float32) + b_ref[0, 0]


def _gmm(xs_sorted, W, b, block_expert, n_blocks, interpret=False):
    grid_spec = pltpu.PrefetchScalarGridSpec(
        num_scalar_prefetch=1,
        grid=(n_blocks,),
        in_specs=[
            pl.BlockSpec((_B, _D), lambda g, be: (g, 0)),
            pl.BlockSpec((1, _D, _D), lambda g, be: (be[g], 0, 0)),
            pl.BlockSpec((1, 1, _D), lambda g, be: (be[g], 0, 0)),
        ],
        out_specs=pl.BlockSpec((_B, _D), lambda g, be: (g, 0)),
    )
    return pl.pallas_call(
        _gmm_body,
        grid_spec=grid_spec,
        out_shape=jax.ShapeDtypeStruct((n_blocks * _B, _D), jnp.float32),
        interpret=interpret,
    )(block_expert, xs_sorted, W, b.reshape(_E, 1, _D))


@jax.jit
def kernel(xs, mxs, actions, W, b):
    n = xs.shape[0]
    n_blocks = n // _B + _E
    n_padded = n_blocks * _B
    src_idx, out_idx, block_expert = _plan(actions, n, n_blocks, n_padded)
    xs_sorted = jnp.take(xs, src_idx, axis=0)
    ys_sorted = _gmm(xs_sorted, W, b, block_expert, n_blocks)
    ys = jnp.take(ys_sorted, out_idx, axis=0)
    return ys, mxs, actions


# speedup vs baseline: 1.2386x; 1.2386x over previous
import functools

import jax
import jax.numpy as jnp
from jax import lax
from jax.experimental import pallas as pl
from jax.experimental.pallas import tpu as pltpu
from jax.experimental.pallas import tpu_sc as plsc

_E = 8
_D = 1024
_B = 256
_NC = 2
_NS = 16
_NW = _NC * _NS
_CHUNK = 32


def _sc_gather(table, idx3d, n_out):
    nchunks = idx3d.shape[1]
    mesh = plsc.VectorSubcoreMesh(core_axis_name="c", subcore_axis_name="s")

    @functools.partial(
        pl.kernel,
        mesh=mesh,
        out_type=jax.ShapeDtypeStruct((n_out, _D), jnp.float32),
        scratch_types=[
            pltpu.VMEM((nchunks, _CHUNK), jnp.int32),
            pltpu.VMEM((2, _CHUNK, _D), jnp.float32),
            pltpu.SemaphoreType.DMA,
            pltpu.SemaphoreType.DMA,
            pltpu.SemaphoreType.DMA,
            pltpu.SemaphoreType.DMA,
        ],
    )
    def gather_kernel(table_hbm, idx_hbm, out_hbm, idx_v, buf, g0, g1, w0, w1):
        wid = lax.axis_index("s") * _NC + lax.axis_index("c")
        row0 = wid * nchunks
        pltpu.sync_copy(idx_hbm.at[wid], idx_v)
        gsem = (g0, g1)
        wsem = (w0, w1)

        def start_gather(c):
            b = c % 2
            return pltpu.async_copy(
                table_hbm.at[idx_v.at[c]], buf.at[b], gsem[b])

        def start_write(c):
            b = c % 2
            dst = out_hbm.at[pl.ds((row0 + c) * _CHUNK, _CHUNK)]
            return pltpu.async_copy(buf.at[b], dst, wsem[b])

        writes = [None, None]
        pending = start_gather(0)
        for c in range(nchunks):
            pending.wait()
            if c + 1 < nchunks:
                nb = (c + 1) % 2
                if writes[nb] is not None:
                    writes[nb].wait()
                    writes[nb] = None
                pending = start_gather(c + 1)
            writes[c % 2] = start_write(c)
        for h in writes:
            if h is not None:
                h.wait()

    return gather_kernel(table, idx3d)


def _gmm_body(be_ref, x_ref, w_ref, b_ref, o_ref):
    o_ref[...] = jax.lax.dot_general(
        x_ref[...], w_ref[0],
        dimension_numbers=(((1,), (1,)), ((), ())),
        preferred_element_type=jnp.float32) + b_ref[0, 0]


def _gmm(xs_sorted, W, b, block_expert, n_blocks):
    grid_spec = pltpu.PrefetchScalarGridSpec(
        num_scalar_prefetch=1,
        grid=(n_blocks,),
        in_specs=[
            pl.BlockSpec((_B, _D), lambda g, be: (g, 0)),
            pl.BlockSpec((1, _D, _D), lambda g, be: (be[g], 0, 0)),
            pl.BlockSpec((1, 1, _D), lambda g, be: (be[g], 0, 0)),
        ],
        out_specs=pl.BlockSpec((_B, _D), lambda g, be: (g, 0)),
    )
    return pl.pallas_call(
        _gmm_body,
        grid_spec=grid_spec,
        out_shape=jax.ShapeDtypeStruct((n_blocks * _B, _D), jnp.float32),
    )(block_expert, xs_sorted, W, b.reshape(_E, 1, _D))


def _plan(actions, n_tokens, n_blocks, n_padded):
    a = actions.astype(jnp.int32)
    ohf = (a[:, None] == jnp.arange(_E, dtype=jnp.int32)[None, :])
    ohf = ohf.astype(jnp.float32)
    csum = jnp.cumsum(ohf, axis=0)
    rank = (jnp.sum(ohf * csum, axis=1) - 1.0).astype(jnp.int32)
    counts = csum[-1].astype(jnp.int32)
    nblk = (counts + _B - 1) // _B
    cum_nblk = jnp.cumsum(nblk)
    blk_start = jnp.concatenate(
        [jnp.zeros((1,), jnp.int32), cum_nblk[:-1]])
    g = jnp.arange(n_blocks, dtype=jnp.int32)
    block_expert = jnp.minimum(
        jnp.sum((g[:, None] >= cum_nblk[None, :]).astype(jnp.int32), axis=1),
        _E - 1).astype(jnp.int32)
    dest = (ohf @ blk_start.astype(jnp.float32)).astype(jnp.int32) * _B + rank
    src = jnp.zeros((n_padded,), jnp.int32).at[dest].set(
        jnp.arange(n_tokens, dtype=jnp.int32))
    return src, dest, block_expert


@jax.jit
def kernel(xs, mxs, actions, W, b):
    n = xs.shape[0]
    n_blocks = n // _B + _E
    n_padded = n_blocks * _B
    src, dest, block_expert = _plan(actions, n, n_blocks, n_padded)
    xs_sorted = _sc_gather(xs, src.reshape(_NW, -1, _CHUNK), n_padded)
    ys_sorted = _gmm(xs_sorted, W, b, block_expert, n_blocks)
    ys = _sc_gather(ys_sorted, dest.reshape(_NW, -1, _CHUNK), n)
    return ys, mxs, actions


# speedup vs baseline: 2.2473x; 1.8144x over previous
import functools

import jax
import jax.numpy as jnp
from jax import lax
from jax.experimental import pallas as pl
from jax.experimental.pallas import tpu as pltpu
from jax.experimental.pallas import tpu_sc as plsc

_E = 8
_D = 1024
_B = 256
_NC = 2
_NS = 16
_NW = _NC * _NS
_CHUNK = 32


def _sc_gather(table, idx3d, n_out):
    nchunks = idx3d.shape[1]
    mesh = plsc.VectorSubcoreMesh(core_axis_name="c", subcore_axis_name="s")

    @functools.partial(
        pl.kernel,
        mesh=mesh,
        out_type=jax.ShapeDtypeStruct((n_out, _D), jnp.float32),
        scratch_types=[
            pltpu.VMEM((nchunks, _CHUNK), jnp.int32),
            pltpu.VMEM((2, _CHUNK, _D), jnp.float32),
            pltpu.SemaphoreType.DMA,
            pltpu.SemaphoreType.DMA,
            pltpu.SemaphoreType.DMA,
            pltpu.SemaphoreType.DMA,
        ],
    )
    def gather_kernel(table_hbm, idx_hbm, out_hbm, idx_v, buf, g0, g1, w0, w1):
        wid = lax.axis_index("s") * _NC + lax.axis_index("c")
        row0 = wid * nchunks
        pltpu.sync_copy(idx_hbm.at[wid], idx_v)
        gsem = (g0, g1)
        wsem = (w0, w1)

        def start_gather(c):
            b = c % 2
            return pltpu.async_copy(
                table_hbm.at[idx_v.at[c]], buf.at[b], gsem[b])

        def start_write(c):
            b = c % 2
            dst = out_hbm.at[pl.ds((row0 + c) * _CHUNK, _CHUNK)]
            return pltpu.async_copy(buf.at[b], dst, wsem[b])

        writes = [None, None]
        pending = start_gather(0)
        for c in range(nchunks):
            pending.wait()
            if c + 1 < nchunks:
                nb = (c + 1) % 2
                if writes[nb] is not None:
                    writes[nb].wait()
                    writes[nb] = None
                pending = start_gather(c + 1)
            writes[c % 2] = start_write(c)
        for h in writes:
            if h is not None:
                h.wait()

    return gather_kernel(table, idx3d)


def _gmm_body(be_ref, x_ref, w_ref, b_ref, o_ref):
    o_ref[...] = jax.lax.dot_general(
        x_ref[...], w_ref[0],
        dimension_numbers=(((1,), (1,)), ((), ())),
        preferred_element_type=jnp.float32) + b_ref[0, 0]


def _gmm(xs_sorted, W, b, block_expert, n_blocks):
    grid_spec = pltpu.PrefetchScalarGridSpec(
        num_scalar_prefetch=1,
        grid=(n_blocks,),
        in_specs=[
            pl.BlockSpec((_B, _D), lambda g, be: (g, 0)),
            pl.BlockSpec((1, _D, _D), lambda g, be: (be[g], 0, 0)),
            pl.BlockSpec((1, 1, _D), lambda g, be: (be[g], 0, 0)),
        ],
        out_specs=pl.BlockSpec((_B, _D), lambda g, be: (g, 0)),
    )
    return pl.pallas_call(
        _gmm_body,
        grid_spec=grid_spec,
        out_shape=jax.ShapeDtypeStruct((n_blocks * _B, _D), jnp.float32),
    )(block_expert, xs_sorted, W, b.reshape(_E, 1, _D))


def _plan(actions, n_tokens, n_blocks, n_padded):
    a = actions.astype(jnp.int32)
    ohf = (a[:, None] == jnp.arange(_E, dtype=jnp.int32)[None, :])
    ohf = ohf.astype(jnp.float32)
    csum = jnp.cumsum(ohf, axis=0)
    rank = (jnp.sum(ohf * csum, axis=1) - 1.0).astype(jnp.int32)
    counts = csum[-1].astype(jnp.int32)
    nblk = (counts + _B - 1) // _B
    cum_nblk = jnp.cumsum(nblk)
    blk_start = jnp.concatenate(
        [jnp.zeros((1,), jnp.int32), cum_nblk[:-1]])
    g = jnp.arange(n_blocks, dtype=jnp.int32)
    block_expert = jnp.minimum(
        jnp.sum((g[:, None] >= cum_nblk[None, :]).astype(jnp.int32), axis=1),
        _E - 1).astype(jnp.int32)
    dest = (ohf @ blk_start.astype(jnp.float32)).astype(jnp.int32) * _B + rank
    pad_base = jnp.arange(n_padded, dtype=jnp.int32) % n_tokens
    src = pad_base.at[dest].set(jnp.arange(n_tokens, dtype=jnp.int32))
    return src, dest, block_expert


@jax.jit
def kernel(xs, mxs, actions, W, b):
    n = xs.shape[0]
    n_blocks = n // _B + _E
    n_padded = n_blocks * _B
    src, dest, block_expert = _plan(actions, n, n_blocks, n_padded)
    xs_sorted = _sc_gather(xs, src.reshape(_NW, -1, _CHUNK), n_padded)
    ys_sorted = _gmm(xs_sorted, W, b, block_expert, n_blocks)
    ys = _sc_gather(ys_sorted, dest.reshape(_NW, -1, _CHUNK), n)
    return ys, mxs, actions
